# baseline (speedup 1.0000x reference)
import jax
import jax.numpy as jnp
from jax.experimental import pallas as pl

WINDOW = 512
HEAD_DIM = 64
NUM_HEADS = 12
BQ = 256
KB = WINDOW + BQ


def _attn_kernel(q_ref, k_ref, v_ref, o_ref):
    i = pl.program_id(1)
    scale = 1.0 / (HEAD_DIM ** 0.5)
    kstart = jnp.maximum(i - 2, 0) * BQ
    q_idx = i * BQ + jax.lax.broadcasted_iota(jnp.int32, (BQ, KB), 0)
    kv_idx = kstart + jax.lax.broadcasted_iota(jnp.int32, (BQ, KB), 1)
    diff = q_idx - kv_idx
    mask = (diff >= 0) & (diff <= WINDOW)
    bias = jnp.where(mask, jnp.float32(0), jnp.float32(float("-inf")))
    for hh in range(NUM_HEADS):
        c0 = hh * HEAD_DIM
        qh = q_ref[0, :, c0:c0 + HEAD_DIM] * scale
        kh = k_ref[0, pl.ds(kstart, KB), c0:c0 + HEAD_DIM]
        vh = v_ref[0, pl.ds(kstart, KB), c0:c0 + HEAD_DIM]
        s = jax.lax.dot_general(
            qh, kh, (((1,), (1,)), ((), ())),
            preferred_element_type=jnp.float32) + bias
        m = jnp.max(s, axis=-1, keepdims=True)
        p = jnp.exp(s - m)
        denom = jnp.sum(p, axis=-1, keepdims=True)
        oh = jax.lax.dot_general(
            p, vh, (((1,), (0,)), ((), ())),
            preferred_element_type=jnp.float32)
        o_ref[0, :, c0:c0 + HEAD_DIM] = oh * (1.0 / denom)


def kernel(qkv):
    b, l, three, h, e = qkv.shape
    ch = h * e
    x = qkv.reshape(b, l, three * ch)
    nq = l // BQ

    out = pl.pallas_call(
        _attn_kernel,
        grid=(b, nq),
        in_specs=[
            pl.BlockSpec((1, BQ, ch), lambda ib, i: (ib, i, 0)),
            pl.BlockSpec((1, l, ch), lambda ib, i: (ib, 0, 1)),
            pl.BlockSpec((1, l, ch), lambda ib, i: (ib, 0, 2)),
        ],
        out_specs=pl.BlockSpec((1, BQ, ch), lambda ib, i: (ib, i, 0)),
        out_shape=jax.ShapeDtypeStruct((b, l, ch), jnp.float32),
    )(x, x, x)

    return out.reshape(b, l, h, e)
